# Initial kernel scaffold; baseline (speedup 1.0000x reference)
import jax
import jax.numpy as jnp
from jax.experimental import pallas as pl


def kernel(x, edge_index, mask_index, fc1_W, fc1_b, W1, b1, W2, b2, W3, b3, W4, b4, W5, b5, W6, b6, fc2_W, fc2_b):
    raise NotImplementedError("write your pallas kernel here")


# speedup vs baseline: 29.6729x; 29.6729x over previous
import functools

import jax
import jax.numpy as jnp
from jax import lax
from jax.experimental import pallas as pl
from jax.experimental.pallas import tpu as pltpu
from jax.experimental.pallas import tpu_sc as plsc

N = 50000
E = 1600000
M = 10000
F = 32

NC = 2
NS = 16
NW = NC * NS

NPAD = 50048
EPT = E // NW
C = 400
NCHUNK = EPT // C
CD = 2000
NDCHUNK = EPT // CD
RPT = NPAD // NS
MPAD = 10240
GPT = MPAD // NW

_mesh = plsc.VectorSubcoreMesh(core_axis_name="c", subcore_axis_name="s")


@functools.partial(
    pl.kernel,
    out_type=jax.ShapeDtypeStruct((NC, NPAD, 16), jnp.float32),
    mesh=_mesh,
    compiler_params=pltpu.CompilerParams(use_tc_tiling_on_sc=False),
    scratch_types=[
        pltpu.VMEM((CD,), jnp.int32),
        pltpu.VMEM((CD, 16), jnp.float32),
        pltpu.VMEM_SHARED((NPAD, 16), jnp.float32),
    ],
)
def _sc_degree(dst_hbm, ones_hbm, zeros1_hbm, out_hbm, dstv, onesv, acc):
    c = lax.axis_index("c")
    s = lax.axis_index("s")
    w = c * NS + s
    r0 = pl.multiple_of(s * RPT, 8)
    pltpu.sync_copy(zeros1_hbm.at[pl.ds(r0, RPT)], acc.at[pl.ds(r0, RPT)])
    pltpu.sync_copy(ones_hbm, onesv)
    plsc.subcore_barrier()
    base = w * EPT

    def body(i, carry):
        off = pl.multiple_of(base + i * CD, 8)
        pltpu.sync_copy(dst_hbm.at[pl.ds(off, CD)], dstv)
        pltpu.sync_copy(onesv, acc.at[dstv], add=True)
        return carry

    lax.fori_loop(0, NDCHUNK, body, 0)
    plsc.subcore_barrier()
    pltpu.sync_copy(acc.at[pl.ds(r0, RPT)], out_hbm.at[c, pl.ds(r0, RPT)])


@functools.partial(
    pl.kernel,
    out_type=jax.ShapeDtypeStruct((NC, NPAD, F), jnp.float32),
    mesh=_mesh,
    compiler_params=pltpu.CompilerParams(use_tc_tiling_on_sc=False),
    scratch_types=[
        pltpu.VMEM((C,), jnp.int32),
        pltpu.VMEM((C,), jnp.int32),
        pltpu.VMEM((C, F), jnp.float32),
        pltpu.VMEM_SHARED((NPAD, F), jnp.float32),
        pltpu.SemaphoreType.DMA,
    ],
)
def _sc_edge_scatter(p_hbm, src_hbm, dst_hbm, zeros_hbm, out_hbm,
                     srcv, dstv, rowsv, acc, sem):
    c = lax.axis_index("c")
    s = lax.axis_index("s")
    w = c * NS + s
    r0 = pl.multiple_of(s * RPT, 8)
    pltpu.sync_copy(zeros_hbm.at[pl.ds(r0, RPT)], acc.at[pl.ds(r0, RPT)])
    plsc.subcore_barrier()
    base = w * EPT

    def body(i, carry):
        off = pl.multiple_of(base + i * C, 8)
        pltpu.sync_copy(src_hbm.at[pl.ds(off, C)], srcv)
        pltpu.sync_copy(dst_hbm.at[pl.ds(off, C)], dstv)
        pltpu.async_copy(p_hbm.at[srcv], rowsv, sem).wait()
        pltpu.sync_copy(rowsv, acc.at[dstv], add=True)
        return carry

    lax.fori_loop(0, NCHUNK, body, 0)
    plsc.subcore_barrier()
    pltpu.sync_copy(acc.at[pl.ds(r0, RPT)], out_hbm.at[c, pl.ds(r0, RPT)])


@functools.partial(
    pl.kernel,
    out_type=jax.ShapeDtypeStruct((MPAD, F), jnp.float32),
    mesh=_mesh,
    compiler_params=pltpu.CompilerParams(use_tc_tiling_on_sc=False),
    scratch_types=[
        pltpu.VMEM((GPT,), jnp.int32),
        pltpu.VMEM((GPT, F), jnp.float32),
        pltpu.SemaphoreType.DMA,
    ],
)
def _sc_mask_gather(h_hbm, mask_hbm, out_hbm, idxv, rowsv, sem):
    c = lax.axis_index("c")
    s = lax.axis_index("s")
    w = c * NS + s
    base = pl.multiple_of(w * GPT, 8)
    pltpu.sync_copy(mask_hbm.at[pl.ds(base, GPT)], idxv)
    pltpu.async_copy(h_hbm.at[idxv], rowsv, sem).wait()
    pltpu.sync_copy(rowsv, out_hbm.at[pl.ds(base, GPT)])


BR = 3128
GRID = NPAD // BR


def _row_valid(blk):
    rid = lax.broadcasted_iota(jnp.int32, (BR, 1), 0) + blk * BR
    return (rid < N).astype(jnp.float32)


def _tc_head_body(deg_ref, x_ref, fc1w_ref, fc1b_ref, w1_ref, dinv_ref, p_ref):
    deg = deg_ref[0, :, 0:1] + deg_ref[1, :, 0:1] + 1.0
    dinv = lax.rsqrt(deg)
    a = jnp.maximum(jnp.dot(x_ref[...], fc1w_ref[...],
                            preferred_element_type=jnp.float32) + fc1b_ref[...], 0.0)
    valid = _row_valid(pl.program_id(0))
    dinv_ref[...] = dinv
    p_ref[...] = dinv * jnp.dot(a, w1_ref[...],
                                preferred_element_type=jnp.float32) * valid


def _tc_mid_body(s_ref, p_ref, dinv_ref, b_ref, wn_ref, pn_ref):
    dinv = dinv_ref[...]
    a = jnp.maximum(dinv * (s_ref[0] + s_ref[1] + p_ref[...]) + b_ref[...], 0.0)
    valid = _row_valid(pl.program_id(0))
    pn_ref[...] = dinv * jnp.dot(a, wn_ref[...],
                                 preferred_element_type=jnp.float32) * valid


def _tc_tail_body(s_ref, p_ref, dinv_ref, b_ref, h_ref):
    h_ref[...] = dinv_ref[...] * (s_ref[0] + s_ref[1] + p_ref[...]) + b_ref[...]


def _tc_fc2_body(rows_ref, fc2w_ref, fc2b_ref, out_ref):
    a = jnp.maximum(rows_ref[...], 0.0)
    out_ref[...] = jnp.dot(a, fc2w_ref[...],
                           preferred_element_type=jnp.float32) + fc2b_ref[...]


def _full(shape):
    return pl.BlockSpec(shape, lambda i: tuple(0 for _ in shape))


_row_spec = pl.BlockSpec((BR, F), lambda i: (i, 0))
_deg_spec = pl.BlockSpec((NC, BR, 16), lambda i: (0, i, 0))
_s_spec = pl.BlockSpec((NC, BR, F), lambda i: (0, i, 0))
_dinv_spec = pl.BlockSpec((BR, 1), lambda i: (i, 0))

_tc_head = pl.pallas_call(
    _tc_head_body,
    grid=(GRID,),
    in_specs=[
        _deg_spec,
        pl.BlockSpec((BR, 4), lambda i: (i, 0)),
        _full((4, F)),
        _full((1, F)),
        _full((F, F)),
    ],
    out_specs=[_dinv_spec, _row_spec],
    out_shape=[
        jax.ShapeDtypeStruct((NPAD, 1), jnp.float32),
        jax.ShapeDtypeStruct((NPAD, F), jnp.float32),
    ],
)

_tc_mid = pl.pallas_call(
    _tc_mid_body,
    grid=(GRID,),
    in_specs=[_s_spec, _row_spec, _dinv_spec, _full((1, F)), _full((F, F))],
    out_specs=_row_spec,
    out_shape=jax.ShapeDtypeStruct((NPAD, F), jnp.float32),
)

_tc_tail = pl.pallas_call(
    _tc_tail_body,
    grid=(GRID,),
    in_specs=[_s_spec, _row_spec, _dinv_spec, _full((1, F))],
    out_specs=_row_spec,
    out_shape=jax.ShapeDtypeStruct((NPAD, F), jnp.float32),
)

_tc_fc2 = pl.pallas_call(
    _tc_fc2_body,
    grid=(MPAD // 2048,),
    in_specs=[
        pl.BlockSpec((2048, F), lambda i: (i, 0)),
        _full((F, 1)),
        _full((1, 1)),
    ],
    out_specs=pl.BlockSpec((2048, 1), lambda i: (i, 0)),
    out_shape=jax.ShapeDtypeStruct((MPAD, 1), jnp.float32),
)


def kernel(x, edge_index, mask_index, fc1_W, fc1_b, W1, b1, W2, b2, W3, b3,
           W4, b4, W5, b5, W6, b6, fc2_W, fc2_b):
    f32 = jnp.float32
    src = edge_index[0]
    dst = edge_index[1]
    xp = jnp.zeros((NPAD, 4), f32).at[:N].set(x)
    mask_pad = jnp.zeros((MPAD,), jnp.int32).at[:M].set(mask_index)
    zeros2 = jnp.zeros((NPAD, F), f32)
    zeros1 = jnp.zeros((NPAD, 16), f32)
    ones1 = jnp.ones((CD, 16), f32)

    deg2 = _sc_degree(dst, ones1, zeros1)
    dinv, p = _tc_head(deg2, xp, fc1_W, fc1_b.reshape(1, F), W1)

    weights = [(W2, b1), (W3, b2), (W4, b3), (W5, b4), (W6, b5)]
    for Wn, b in weights:
        s2 = _sc_edge_scatter(p, src, dst, zeros2)
        p = _tc_mid(s2, p, dinv, b.reshape(1, F), Wn)

    s2 = _sc_edge_scatter(p, src, dst, zeros2)
    h = _tc_tail(s2, p, dinv, b6.reshape(1, F))

    rows = _sc_mask_gather(h, mask_pad)
    y = _tc_fc2(rows, fc2_W, fc2_b.reshape(1, 1))
    return y[:M]


# speedup vs baseline: 42.9493x; 1.4474x over previous
import functools

import jax
import jax.numpy as jnp
from jax import lax
from jax.experimental import pallas as pl
from jax.experimental.pallas import tpu as pltpu
from jax.experimental.pallas import tpu_sc as plsc

N = 50000
E = 1600000
M = 10000
F = 32

NC = 2
NS = 16
NW = NC * NS

NPAD = 50048
EPT = E // NW
C = 400
NCHUNK = EPT // C
CD = 2000
NDCHUNK = EPT // CD
RPT = NPAD // NS
MPAD = 10240
GPT = MPAD // NW

_mesh = plsc.VectorSubcoreMesh(core_axis_name="c", subcore_axis_name="s")


@functools.partial(
    pl.kernel,
    out_type=jax.ShapeDtypeStruct((NC, NPAD, 16), jnp.float32),
    mesh=_mesh,
    compiler_params=pltpu.CompilerParams(use_tc_tiling_on_sc=False),
    scratch_types=[
        pltpu.VMEM((CD,), jnp.int32),
        pltpu.VMEM((CD, 16), jnp.float32),
        pltpu.VMEM_SHARED((NPAD, 16), jnp.float32),
    ],
)
def _sc_degree(dst_hbm, ones_hbm, zeros1_hbm, out_hbm, dstv, onesv, acc):
    c = lax.axis_index("c")
    s = lax.axis_index("s")
    w = c * NS + s
    r0 = pl.multiple_of(s * RPT, 8)
    pltpu.sync_copy(zeros1_hbm.at[pl.ds(r0, RPT)], acc.at[pl.ds(r0, RPT)])
    pltpu.sync_copy(ones_hbm, onesv)
    plsc.subcore_barrier()
    base = w * EPT

    def body(i, carry):
        off = pl.multiple_of(base + i * CD, 8)
        pltpu.sync_copy(dst_hbm.at[pl.ds(off, CD)], dstv)
        pltpu.sync_copy(onesv, acc.at[dstv], add=True)
        return carry

    lax.fori_loop(0, NDCHUNK, body, 0)
    plsc.subcore_barrier()
    pltpu.sync_copy(acc.at[pl.ds(r0, RPT)], out_hbm.at[c, pl.ds(r0, RPT)])


@functools.partial(
    pl.kernel,
    out_type=jax.ShapeDtypeStruct((NC, NPAD, F), jnp.float32),
    mesh=_mesh,
    compiler_params=pltpu.CompilerParams(use_tc_tiling_on_sc=False),
    scratch_types=[
        pltpu.VMEM((C,), jnp.int32),
        pltpu.VMEM((C,), jnp.int32),
        pltpu.VMEM((C, F), jnp.float32),
        pltpu.VMEM((C,), jnp.int32),
        pltpu.VMEM((C,), jnp.int32),
        pltpu.VMEM((C, F), jnp.float32),
        pltpu.VMEM_SHARED((NPAD, F), jnp.float32),
        pltpu.SemaphoreType.DMA,
        pltpu.SemaphoreType.DMA,
        pltpu.SemaphoreType.DMA,
        pltpu.SemaphoreType.DMA,
        pltpu.SemaphoreType.DMA,
    ],
)
def _sc_edge_scatter(p_hbm, src_hbm, dst_hbm, zeros_hbm, out_hbm,
                     src0, dst0, rows0, src1, dst1, rows1,
                     acc, sidx0, sidx1, sg, ss0, ss1):
    c = lax.axis_index("c")
    s = lax.axis_index("s")
    w = c * NS + s
    r0 = pl.multiple_of(s * RPT, 8)
    pltpu.sync_copy(zeros_hbm.at[pl.ds(r0, RPT)], acc.at[pl.ds(r0, RPT)])
    plsc.subcore_barrier()
    base = w * EPT

    def idx_load(g, srcb, dstb, sem):
        off = pl.multiple_of(base + g * C, 8)
        pltpu.async_copy(src_hbm.at[pl.ds(off, C)], srcb, sem)
        pltpu.async_copy(dst_hbm.at[pl.ds(off, C)], dstb, sem)

    def wait_idx(srcb, dstb, sem):
        pltpu.make_async_copy(src_hbm.at[pl.ds(0, C)], srcb, sem).wait()
        pltpu.make_async_copy(dst_hbm.at[pl.ds(0, C)], dstb, sem).wait()

    def gather_scatter(srcb, dstb, rowsb, sidx, ssem):
        wait_idx(srcb, dstb, sidx)
        pltpu.async_copy(p_hbm.at[srcb], rowsb, sg).wait()
        pltpu.async_copy(rowsb, acc.at[dstb], ssem, add=True)

    def wait_scatter(rowsb, dstb, ssem):
        pltpu.make_async_copy(rowsb, acc.at[dstb], ssem).wait()

    idx_load(0, src0, dst0, sidx0)
    idx_load(1, src1, dst1, sidx1)
    gather_scatter(src0, dst0, rows0, sidx0, ss0)
    gather_scatter(src1, dst1, rows1, sidx1, ss1)

    def body(i, carry):
        g = 2 * i
        wait_scatter(rows0, dst0, ss0)
        idx_load(g, src0, dst0, sidx0)
        wait_scatter(rows1, dst1, ss1)
        idx_load(g + 1, src1, dst1, sidx1)
        gather_scatter(src0, dst0, rows0, sidx0, ss0)
        gather_scatter(src1, dst1, rows1, sidx1, ss1)
        return carry

    lax.fori_loop(1, NCHUNK // 2, body, 0)
    wait_scatter(rows0, dst0, ss0)
    idx_load(NCHUNK - 1, src0, dst0, sidx0)
    wait_scatter(rows1, dst1, ss1)
    gather_scatter(src0, dst0, rows0, sidx0, ss0)
    wait_scatter(rows0, dst0, ss0)
    plsc.subcore_barrier()
    pltpu.sync_copy(acc.at[pl.ds(r0, RPT)], out_hbm.at[c, pl.ds(r0, RPT)])


@functools.partial(
    pl.kernel,
    out_type=jax.ShapeDtypeStruct((MPAD, F), jnp.float32),
    mesh=_mesh,
    compiler_params=pltpu.CompilerParams(use_tc_tiling_on_sc=False),
    scratch_types=[
        pltpu.VMEM((GPT,), jnp.int32),
        pltpu.VMEM((GPT, F), jnp.float32),
        pltpu.SemaphoreType.DMA,
    ],
)
def _sc_mask_gather(h_hbm, mask_hbm, out_hbm, idxv, rowsv, sem):
    c = lax.axis_index("c")
    s = lax.axis_index("s")
    w = c * NS + s
    base = pl.multiple_of(w * GPT, 8)
    pltpu.sync_copy(mask_hbm.at[pl.ds(base, GPT)], idxv)
    pltpu.async_copy(h_hbm.at[idxv], rowsv, sem).wait()
    pltpu.sync_copy(rowsv, out_hbm.at[pl.ds(base, GPT)])


BR = 3128
GRID = NPAD // BR


def _row_valid(blk):
    rid = lax.broadcasted_iota(jnp.int32, (BR, 1), 0) + blk * BR
    return (rid < N).astype(jnp.float32)


def _tc_head_body(deg_ref, x_ref, fc1w_ref, fc1b_ref, w1_ref, dinv_ref, p_ref):
    deg = deg_ref[0, :, 0:1] + deg_ref[1, :, 0:1] + 1.0
    dinv = lax.rsqrt(deg)
    a = jnp.maximum(jnp.dot(x_ref[...], fc1w_ref[...],
                            preferred_element_type=jnp.float32) + fc1b_ref[...], 0.0)
    valid = _row_valid(pl.program_id(0))
    dinv_ref[...] = dinv
    p_ref[...] = dinv * jnp.dot(a, w1_ref[...],
                                preferred_element_type=jnp.float32) * valid


def _tc_mid_body(s_ref, p_ref, dinv_ref, b_ref, wn_ref, pn_ref):
    dinv = dinv_ref[...]
    a = jnp.maximum(dinv * (s_ref[0] + s_ref[1] + p_ref[...]) + b_ref[...], 0.0)
    valid = _row_valid(pl.program_id(0))
    pn_ref[...] = dinv * jnp.dot(a, wn_ref[...],
                                 preferred_element_type=jnp.float32) * valid


def _tc_tail_body(s_ref, p_ref, dinv_ref, b_ref, h_ref):
    h_ref[...] = dinv_ref[...] * (s_ref[0] + s_ref[1] + p_ref[...]) + b_ref[...]


def _tc_fc2_body(rows_ref, fc2w_ref, fc2b_ref, out_ref):
    a = jnp.maximum(rows_ref[...], 0.0)
    out_ref[...] = jnp.dot(a, fc2w_ref[...],
                           preferred_element_type=jnp.float32) + fc2b_ref[...]


def _full(shape):
    return pl.BlockSpec(shape, lambda i: tuple(0 for _ in shape))


_row_spec = pl.BlockSpec((BR, F), lambda i: (i, 0))
_deg_spec = pl.BlockSpec((NC, BR, 16), lambda i: (0, i, 0))
_s_spec = pl.BlockSpec((NC, BR, F), lambda i: (0, i, 0))
_dinv_spec = pl.BlockSpec((BR, 1), lambda i: (i, 0))

_tc_head = pl.pallas_call(
    _tc_head_body,
    grid=(GRID,),
    in_specs=[
        _deg_spec,
        pl.BlockSpec((BR, 4), lambda i: (i, 0)),
        _full((4, F)),
        _full((1, F)),
        _full((F, F)),
    ],
    out_specs=[_dinv_spec, _row_spec],
    out_shape=[
        jax.ShapeDtypeStruct((NPAD, 1), jnp.float32),
        jax.ShapeDtypeStruct((NPAD, F), jnp.float32),
    ],
)

_tc_mid = pl.pallas_call(
    _tc_mid_body,
    grid=(GRID,),
    in_specs=[_s_spec, _row_spec, _dinv_spec, _full((1, F)), _full((F, F))],
    out_specs=_row_spec,
    out_shape=jax.ShapeDtypeStruct((NPAD, F), jnp.float32),
)

_tc_tail = pl.pallas_call(
    _tc_tail_body,
    grid=(GRID,),
    in_specs=[_s_spec, _row_spec, _dinv_spec, _full((1, F))],
    out_specs=_row_spec,
    out_shape=jax.ShapeDtypeStruct((NPAD, F), jnp.float32),
)

_tc_fc2 = pl.pallas_call(
    _tc_fc2_body,
    grid=(MPAD // 2048,),
    in_specs=[
        pl.BlockSpec((2048, F), lambda i: (i, 0)),
        _full((F, 1)),
        _full((1, 1)),
    ],
    out_specs=pl.BlockSpec((2048, 1), lambda i: (i, 0)),
    out_shape=jax.ShapeDtypeStruct((MPAD, 1), jnp.float32),
)


def kernel(x, edge_index, mask_index, fc1_W, fc1_b, W1, b1, W2, b2, W3, b3,
           W4, b4, W5, b5, W6, b6, fc2_W, fc2_b):
    f32 = jnp.float32
    src = edge_index[0]
    dst = edge_index[1]
    xp = jnp.zeros((NPAD, 4), f32).at[:N].set(x)
    mask_pad = jnp.zeros((MPAD,), jnp.int32).at[:M].set(mask_index)
    zeros2 = jnp.zeros((NPAD, F), f32)
    zeros1 = jnp.zeros((NPAD, 16), f32)
    ones1 = jnp.ones((CD, 16), f32)

    deg2 = _sc_degree(dst, ones1, zeros1)
    dinv, p = _tc_head(deg2, xp, fc1_W, fc1_b.reshape(1, F), W1)

    weights = [(W2, b1), (W3, b2), (W4, b3), (W5, b4), (W6, b5)]
    for Wn, b in weights:
        s2 = _sc_edge_scatter(p, src, dst, zeros2)
        p = _tc_mid(s2, p, dinv, b.reshape(1, F), Wn)

    s2 = _sc_edge_scatter(p, src, dst, zeros2)
    h = _tc_tail(s2, p, dinv, b6.reshape(1, F))

    rows = _sc_mask_gather(h, mask_pad)
    y = _tc_fc2(rows, fc2_W, fc2_b.reshape(1, 1))
    return y[:M]


# speedup vs baseline: 58.1251x; 1.3533x over previous
import functools

import jax
import jax.numpy as jnp
from jax import lax
from jax.experimental import pallas as pl
from jax.experimental.pallas import tpu as pltpu
from jax.experimental.pallas import tpu_sc as plsc

N = 50000
E = 1600000
M = 10000
F = 32

NC = 2
NS = 16
NW = NC * NS

NPAD = 50048
EPT = E // NW
C = 400
NCHUNK = EPT // C
RPT = NPAD // NS
MPAD = 10240
GPT = MPAD // NW

RW = NPAD * F // 128

_mesh = plsc.VectorSubcoreMesh(core_axis_name="c", subcore_axis_name="s")
_sc_params = pltpu.CompilerParams(use_tc_tiling_on_sc=False)


@functools.partial(
    pl.kernel,
    out_type=jax.ShapeDtypeStruct((NC, NPAD, F), jnp.float32),
    mesh=_mesh,
    compiler_params=_sc_params,
    scratch_types=[
        pltpu.VMEM((C,), jnp.int32),
        pltpu.VMEM((C,), jnp.int32),
        pltpu.VMEM((C, F), jnp.float32),
        pltpu.SemaphoreType.DMA,
        pltpu.SemaphoreType.DMA,
        pltpu.VMEM_SHARED((NPAD, F), jnp.float32),
        pltpu.SemaphoreType.DMA,
        pltpu.SemaphoreType.DMA,
        pltpu.SemaphoreType.DMA,
    ],
)
def _sc_degree(dst_hbm, ones_hbm, zeros_hbm, out_hbm,
               dst0, dst1, onesv, sidx0, sidx1, acc, sg, ss0, ss1):
    c = lax.axis_index("c")
    s = lax.axis_index("s")
    w = c * NS + s
    r0 = pl.multiple_of(s * RPT, 8)
    pltpu.sync_copy(zeros_hbm.at[pl.ds(r0, RPT)], acc.at[pl.ds(r0, RPT)])
    pltpu.sync_copy(ones_hbm, onesv)
    plsc.subcore_barrier()
    base = w * EPT

    def idx_load(g, dstb, sem):
        off = pl.multiple_of(base + g * C, 8)
        pltpu.async_copy(dst_hbm.at[pl.ds(off, C)], dstb, sem)

    def scat(dstb, sidx, ssem):
        pltpu.make_async_copy(dst_hbm.at[pl.ds(0, C)], dstb, sidx).wait()
        pltpu.async_copy(onesv, acc.at[dstb], ssem, add=True)

    def wait_scat(dstb, ssem):
        pltpu.make_async_copy(onesv, acc.at[dstb], ssem).wait()

    idx_load(0, dst0, sidx0)
    idx_load(1, dst1, sidx1)
    scat(dst0, sidx0, ss0)
    scat(dst1, sidx1, ss1)

    def body(i, carry):
        g = 2 * i
        wait_scat(dst0, ss0)
        idx_load(g, dst0, sidx0)
        wait_scat(dst1, ss1)
        idx_load(g + 1, dst1, sidx1)
        scat(dst0, sidx0, ss0)
        scat(dst1, sidx1, ss1)
        return carry

    lax.fori_loop(1, NCHUNK // 2, body, 0)
    wait_scat(dst0, ss0)
    idx_load(NCHUNK - 1, dst0, sidx0)
    wait_scat(dst1, ss1)
    scat(dst0, sidx0, ss0)
    wait_scat(dst0, ss0)
    plsc.subcore_barrier()
    pltpu.sync_copy(acc.at[pl.ds(r0, RPT)], out_hbm.at[c, pl.ds(r0, RPT)])


@functools.partial(
    pl.kernel,
    out_type=jax.ShapeDtypeStruct((NC, NPAD, F), jnp.float32),
    mesh=_mesh,
    compiler_params=_sc_params,
    scratch_types=[
        pltpu.VMEM((C,), jnp.int32),
        pltpu.VMEM((C,), jnp.int32),
        pltpu.VMEM((C, F), jnp.float32),
        pltpu.VMEM((C,), jnp.int32),
        pltpu.VMEM((C,), jnp.int32),
        pltpu.VMEM((C, F), jnp.float32),
        pltpu.VMEM_SHARED((NPAD, F), jnp.float32),
        pltpu.SemaphoreType.DMA,
        pltpu.SemaphoreType.DMA,
        pltpu.SemaphoreType.DMA,
        pltpu.SemaphoreType.DMA,
        pltpu.SemaphoreType.DMA,
    ],
)
def _sc_edge_scatter(p_hbm, src_hbm, dst_hbm, zeros_hbm, out_hbm,
                     src0, dst0, rows0, src1, dst1, rows1,
                     acc, sidx0, sidx1, sg, ss0, ss1):
    c = lax.axis_index("c")
    s = lax.axis_index("s")
    w = c * NS + s
    r0 = pl.multiple_of(s * RPT, 8)
    pltpu.sync_copy(zeros_hbm.at[pl.ds(r0, RPT)], acc.at[pl.ds(r0, RPT)])
    plsc.subcore_barrier()
    base = w * EPT

    def idx_load(g, srcb, dstb, sem):
        off = pl.multiple_of(base + g * C, 8)
        pltpu.async_copy(src_hbm.at[pl.ds(off, C)], srcb, sem)
        pltpu.async_copy(dst_hbm.at[pl.ds(off, C)], dstb, sem)

    def wait_idx(srcb, dstb, sem):
        pltpu.make_async_copy(src_hbm.at[pl.ds(0, C)], srcb, sem).wait()
        pltpu.make_async_copy(dst_hbm.at[pl.ds(0, C)], dstb, sem).wait()

    def gather_scatter(srcb, dstb, rowsb, sidx, ssem):
        wait_idx(srcb, dstb, sidx)
        pltpu.async_copy(p_hbm.at[srcb], rowsb, sg).wait()
        pltpu.async_copy(rowsb, acc.at[dstb], ssem, add=True)

    def wait_scatter(rowsb, dstb, ssem):
        pltpu.make_async_copy(rowsb, acc.at[dstb], ssem).wait()

    idx_load(0, src0, dst0, sidx0)
    idx_load(1, src1, dst1, sidx1)
    gather_scatter(src0, dst0, rows0, sidx0, ss0)
    gather_scatter(src1, dst1, rows1, sidx1, ss1)

    def body(i, carry):
        g = 2 * i
        wait_scatter(rows0, dst0, ss0)
        idx_load(g, src0, dst0, sidx0)
        wait_scatter(rows1, dst1, ss1)
        idx_load(g + 1, src1, dst1, sidx1)
        gather_scatter(src0, dst0, rows0, sidx0, ss0)
        gather_scatter(src1, dst1, rows1, sidx1, ss1)
        return carry

    lax.fori_loop(1, NCHUNK // 2, body, 0)
    wait_scatter(rows0, dst0, ss0)
    idx_load(NCHUNK - 1, src0, dst0, sidx0)
    wait_scatter(rows1, dst1, ss1)
    gather_scatter(src0, dst0, rows0, sidx0, ss0)
    wait_scatter(rows0, dst0, ss0)
    plsc.subcore_barrier()
    pltpu.sync_copy(acc.at[pl.ds(r0, RPT)], out_hbm.at[c, pl.ds(r0, RPT)])


@functools.partial(
    pl.kernel,
    out_type=jax.ShapeDtypeStruct((MPAD, F), jnp.float32),
    mesh=_mesh,
    compiler_params=_sc_params,
    scratch_types=[
        pltpu.VMEM((GPT,), jnp.int32),
        pltpu.VMEM((GPT, F), jnp.float32),
        pltpu.SemaphoreType.DMA,
    ],
)
def _sc_mask_gather(h_hbm, mask_hbm, out_hbm, idxv, rowsv, sem):
    c = lax.axis_index("c")
    s = lax.axis_index("s")
    w = c * NS + s
    base = pl.multiple_of(w * GPT, 8)
    pltpu.sync_copy(mask_hbm.at[pl.ds(base, GPT)], idxv)
    pltpu.async_copy(h_hbm.at[idxv], rowsv, sem).wait()
    pltpu.sync_copy(rowsv, out_hbm.at[pl.ds(base, GPT)])


BRW = 3128
GRIDW = RW // BRW


def _tc_head_body(deg_ref, xq_ref, fc1w_ref, fc1b_ref, w1_ref,
                  dinv_ref, p_ref):
    deg = deg_ref[0] + deg_ref[1] + 1.0
    dinv = lax.rsqrt(deg)
    a = jnp.maximum(jnp.dot(xq_ref[...], fc1w_ref[...],
                            preferred_element_type=jnp.float32) + fc1b_ref[...], 0.0)
    dinv_ref[...] = dinv
    p_ref[...] = dinv * jnp.dot(a, w1_ref[...],
                                preferred_element_type=jnp.float32)


def _tc_mid_body(s_ref, p_ref, dinv_ref, b_ref, wn_ref, pn_ref):
    dinv = dinv_ref[...]
    a = jnp.maximum(dinv * (s_ref[0] + s_ref[1] + p_ref[...]) + b_ref[...], 0.0)
    pn_ref[...] = dinv * jnp.dot(a, wn_ref[...],
                                 preferred_element_type=jnp.float32)


def _tc_tail_body(s_ref, p_ref, dinv_ref, b_ref, h_ref):
    h_ref[...] = dinv_ref[...] * (s_ref[0] + s_ref[1] + p_ref[...]) + b_ref[...]


def _tc_fc2_body(rows_ref, fc2w_ref, fc2b_ref, out_ref):
    a = jnp.maximum(rows_ref[...], 0.0)
    out_ref[...] = jnp.dot(a, fc2w_ref[...],
                           preferred_element_type=jnp.float32) + fc2b_ref[...]


def _full(shape):
    return pl.BlockSpec(shape, lambda i: tuple(0 for _ in shape))


_roww_spec = pl.BlockSpec((BRW, 128), lambda i: (i, 0))
_sw_spec = pl.BlockSpec((NC, BRW, 128), lambda i: (0, i, 0))

_tc_head = pl.pallas_call(
    _tc_head_body,
    grid=(GRIDW,),
    in_specs=[
        _sw_spec,
        pl.BlockSpec((BRW, 16), lambda i: (i, 0)),
        _full((16, 128)),
        _full((1, 128)),
        _full((128, 128)),
    ],
    out_specs=[_roww_spec, _roww_spec],
    out_shape=[
        jax.ShapeDtypeStruct((RW, 128), jnp.float32),
        jax.ShapeDtypeStruct((RW, 128), jnp.float32),
    ],
)

_tc_mid = pl.pallas_call(
    _tc_mid_body,
    grid=(GRIDW,),
    in_specs=[_sw_spec, _roww_spec, _roww_spec, _full((1, 128)), _full((128, 128))],
    out_specs=_roww_spec,
    out_shape=jax.ShapeDtypeStruct((RW, 128), jnp.float32),
)

_tc_tail = pl.pallas_call(
    _tc_tail_body,
    grid=(GRIDW,),
    in_specs=[_sw_spec, _roww_spec, _roww_spec, _full((1, 128))],
    out_specs=_roww_spec,
    out_shape=jax.ShapeDtypeStruct((RW, 128), jnp.float32),
)

_tc_fc2 = pl.pallas_call(
    _tc_fc2_body,
    grid=(),
    in_specs=[
        pl.BlockSpec((MPAD // 4, 128), lambda: (0, 0)),
        pl.BlockSpec((128, 4), lambda: (0, 0)),
        pl.BlockSpec((1, 4), lambda: (0, 0)),
    ],
    out_specs=pl.BlockSpec((MPAD // 4, 4), lambda: (0, 0)),
    out_shape=jax.ShapeDtypeStruct((MPAD // 4, 4), jnp.float32),
)


def kernel(x, edge_index, mask_index, fc1_W, fc1_b, W1, b1, W2, b2, W3, b3,
           W4, b4, W5, b5, W6, b6, fc2_W, fc2_b):
    f32 = jnp.float32
    eye4 = jnp.eye(4, dtype=f32)
    xq = jnp.zeros((NPAD, 4), f32).at[:N].set(x).reshape(NPAD // 4, 16)
    mask_pad = jnp.zeros((MPAD,), jnp.int32).at[:M].set(mask_index)
    zeros2 = jnp.zeros((NPAD, F), f32)
    ones2 = jnp.ones((C, F), f32)

    fc1blk = jnp.kron(eye4, fc1_W)
    fc1b_w = jnp.tile(fc1_b, 4).reshape(1, 128)
    wblk = [jnp.kron(eye4, Wn) for Wn in (W1, W2, W3, W4, W5, W6)]
    bw = [jnp.tile(bn, 4).reshape(1, 128) for bn in (b1, b2, b3, b4, b5, b6)]
    fc2blk = jnp.kron(eye4, fc2_W)

    src_i = edge_index[0]
    dst_i = edge_index[1]
    deg2 = _sc_degree(dst_i, ones2, zeros2)
    deg2w = deg2.reshape(NC, RW, 128)
    dinv, p = _tc_head(deg2w, xq, fc1blk, fc1b_w, wblk[0])

    for i in range(5):
        s2 = _sc_edge_scatter(p.reshape(NPAD, F), src_i, dst_i, zeros2)
        p = _tc_mid(s2.reshape(NC, RW, 128), p, dinv, bw[i], wblk[i + 1])

    s2 = _sc_edge_scatter(p.reshape(NPAD, F), src_i, dst_i, zeros2)
    h = _tc_tail(s2.reshape(NC, RW, 128), p, dinv, bw[5])

    rows = _sc_mask_gather(h.reshape(NPAD, F), mask_pad)
    y4 = _tc_fc2(rows.reshape(MPAD // 4, 128), fc2blk, fc2_b.reshape(1, 1) * jnp.ones((1, 4), f32))
    return y4.reshape(MPAD, 1)[:M]


# speedup vs baseline: 72.5120x; 1.2475x over previous
import functools

import jax
import jax.numpy as jnp
from jax import lax
from jax.experimental import pallas as pl
from jax.experimental.pallas import tpu as pltpu
from jax.experimental.pallas import tpu_sc as plsc

N = 50000
E = 1600000
M = 10000
F = 32

NC = 2
NS = 16
NW = NC * NS

NPAD = 50048
EPT = E // NW
C = 400
NCHUNK = EPT // C
CE = 200
NCHUNKE = EPT // CE
assert NCHUNKE % 4 == 2
RPT = NPAD // NS
MPAD = 10240
GPT = MPAD // NW

RW = NPAD * F // 128

_mesh = plsc.VectorSubcoreMesh(core_axis_name="c", subcore_axis_name="s")
_sc_params = pltpu.CompilerParams(use_tc_tiling_on_sc=False)


@functools.partial(
    pl.kernel,
    out_type=jax.ShapeDtypeStruct((NC, NPAD, F), jnp.float32),
    mesh=_mesh,
    compiler_params=_sc_params,
    scratch_types=[
        pltpu.VMEM((C,), jnp.int32),
        pltpu.VMEM((C,), jnp.int32),
        pltpu.VMEM((C, F), jnp.float32),
        pltpu.SemaphoreType.DMA,
        pltpu.SemaphoreType.DMA,
        pltpu.VMEM_SHARED((NPAD, F), jnp.float32),
        pltpu.SemaphoreType.DMA,
        pltpu.SemaphoreType.DMA,
        pltpu.SemaphoreType.DMA,
    ],
)
def _sc_degree(dst_hbm, ones_hbm, zeros_hbm, out_hbm,
               dst0, dst1, onesv, sidx0, sidx1, acc, sg, ss0, ss1):
    c = lax.axis_index("c")
    s = lax.axis_index("s")
    w = c * NS + s
    r0 = pl.multiple_of(s * RPT, 8)
    pltpu.sync_copy(zeros_hbm.at[pl.ds(r0, RPT)], acc.at[pl.ds(r0, RPT)])
    pltpu.sync_copy(ones_hbm, onesv)
    plsc.subcore_barrier()
    base = w * EPT

    def idx_load(g, dstb, sem):
        off = pl.multiple_of(base + g * C, 8)
        pltpu.async_copy(dst_hbm.at[pl.ds(off, C)], dstb, sem)

    def scat(dstb, sidx, ssem):
        pltpu.make_async_copy(dst_hbm.at[pl.ds(0, C)], dstb, sidx).wait()
        pltpu.async_copy(onesv, acc.at[dstb], ssem, add=True)

    def wait_scat(dstb, ssem):
        pltpu.make_async_copy(onesv, acc.at[dstb], ssem).wait()

    idx_load(0, dst0, sidx0)
    idx_load(1, dst1, sidx1)
    scat(dst0, sidx0, ss0)
    scat(dst1, sidx1, ss1)

    def body(i, carry):
        g = 2 * i
        wait_scat(dst0, ss0)
        idx_load(g, dst0, sidx0)
        wait_scat(dst1, ss1)
        idx_load(g + 1, dst1, sidx1)
        scat(dst0, sidx0, ss0)
        scat(dst1, sidx1, ss1)
        return carry

    lax.fori_loop(1, NCHUNK // 2, body, 0)
    wait_scat(dst0, ss0)
    idx_load(NCHUNK - 1, dst0, sidx0)
    wait_scat(dst1, ss1)
    scat(dst0, sidx0, ss0)
    wait_scat(dst0, ss0)
    plsc.subcore_barrier()
    pltpu.sync_copy(acc.at[pl.ds(r0, RPT)], out_hbm.at[c, pl.ds(r0, RPT)])


NB = 4


@functools.partial(
    pl.kernel,
    out_type=jax.ShapeDtypeStruct((NC, NPAD, F), jnp.float32),
    mesh=_mesh,
    compiler_params=_sc_params,
    scratch_types=[
        [pltpu.VMEM((CE,), jnp.int32) for _ in range(NB)],
        [pltpu.VMEM((CE,), jnp.int32) for _ in range(NB)],
        [pltpu.VMEM((CE, F), jnp.float32) for _ in range(NB)],
        pltpu.VMEM_SHARED((NPAD, F), jnp.float32),
        [pltpu.SemaphoreType.DMA for _ in range(NB)],
        [pltpu.SemaphoreType.DMA for _ in range(NB)],
        [pltpu.SemaphoreType.DMA for _ in range(NB)],
    ],
)
def _sc_edge_scatter(p_hbm, src_hbm, dst_hbm, zeros_hbm, out_hbm,
                     srcb, dstb, rowsb, acc, sidx, sg, ss):
    c = lax.axis_index("c")
    s = lax.axis_index("s")
    w = c * NS + s
    r0 = pl.multiple_of(s * RPT, 8)
    pltpu.sync_copy(zeros_hbm.at[pl.ds(r0, RPT)], acc.at[pl.ds(r0, RPT)])
    plsc.subcore_barrier()
    base = w * EPT

    def I(g, b):
        off = pl.multiple_of(base + g * CE, 8)
        pltpu.async_copy(src_hbm.at[pl.ds(off, CE)], srcb[b], sidx[b])
        pltpu.async_copy(dst_hbm.at[pl.ds(off, CE)], dstb[b], sidx[b])

    def A(b):
        pltpu.make_async_copy(src_hbm.at[pl.ds(0, CE)], srcb[b], sidx[b]).wait()
        pltpu.make_async_copy(dst_hbm.at[pl.ds(0, CE)], dstb[b], sidx[b]).wait()
        pltpu.async_copy(p_hbm.at[srcb[b]], rowsb[b], sg[b])

    def B(b):
        pltpu.make_async_copy(p_hbm.at[srcb[b]], rowsb[b], sg[b]).wait()
        pltpu.async_copy(rowsb[b], acc.at[dstb[b]], ss[b], add=True)

    def WS(b):
        pltpu.make_async_copy(rowsb[b], acc.at[dstb[b]], ss[b]).wait()

    I(0, 0)
    I(1, 1)
    A(0)
    I(2, 2)
    A(1)
    B(0)
    I(3, 3)
    A(2)
    B(1)

    def body(i, carry):
        g0 = 4 * i
        for j in range(4):
            g = g0 + j
            b = j
            b2 = (j + 2) % 4
            b1 = (j + 1) % 4
            WS(b2)
            I(g + 2, b2)
            A(b1)
            B(b)
        return carry

    WS(0)
    I(4, 0)
    A(3)
    B(2)
    WS(1)
    I(5, 1)
    A(0)
    B(3)

    lax.fori_loop(1, (NCHUNKE - 2) // 4, body, 0)

    A(1)
    B(0)
    B(1)
    WS(0)
    WS(1)
    WS(2)
    WS(3)
    plsc.subcore_barrier()
    pltpu.sync_copy(acc.at[pl.ds(r0, RPT)], out_hbm.at[c, pl.ds(r0, RPT)])


@functools.partial(
    pl.kernel,
    out_type=jax.ShapeDtypeStruct((MPAD, F), jnp.float32),
    mesh=_mesh,
    compiler_params=_sc_params,
    scratch_types=[
        pltpu.VMEM((GPT,), jnp.int32),
        pltpu.VMEM((GPT, F), jnp.float32),
        pltpu.SemaphoreType.DMA,
    ],
)
def _sc_mask_gather(h_hbm, mask_hbm, out_hbm, idxv, rowsv, sem):
    c = lax.axis_index("c")
    s = lax.axis_index("s")
    w = c * NS + s
    base = pl.multiple_of(w * GPT, 8)
    pltpu.sync_copy(mask_hbm.at[pl.ds(base, GPT)], idxv)
    pltpu.async_copy(h_hbm.at[idxv], rowsv, sem).wait()
    pltpu.sync_copy(rowsv, out_hbm.at[pl.ds(base, GPT)])


BRW = 3128
GRIDW = RW // BRW


def _tc_head_body(deg_ref, xq_ref, fc1w_ref, fc1b_ref, w1_ref,
                  dinv_ref, p_ref):
    deg = deg_ref[0] + deg_ref[1] + 1.0
    dinv = lax.rsqrt(deg)
    a = jnp.maximum(jnp.dot(xq_ref[...], fc1w_ref[...],
                            preferred_element_type=jnp.float32) + fc1b_ref[...], 0.0)
    dinv_ref[...] = dinv
    p_ref[...] = dinv * jnp.dot(a, w1_ref[...],
                                preferred_element_type=jnp.float32)


def _tc_mid_body(s_ref, p_ref, dinv_ref, b_ref, wn_ref, pn_ref):
    dinv = dinv_ref[...]
    a = jnp.maximum(dinv * (s_ref[0] + s_ref[1] + p_ref[...]) + b_ref[...], 0.0)
    pn_ref[...] = dinv * jnp.dot(a, wn_ref[...],
                                 preferred_element_type=jnp.float32)


def _tc_tail_body(s_ref, p_ref, dinv_ref, b_ref, h_ref):
    h_ref[...] = dinv_ref[...] * (s_ref[0] + s_ref[1] + p_ref[...]) + b_ref[...]


def _tc_fc2_body(rows_ref, fc2w_ref, fc2b_ref, out_ref):
    a = jnp.maximum(rows_ref[...], 0.0)
    out_ref[...] = jnp.dot(a, fc2w_ref[...],
                           preferred_element_type=jnp.float32) + fc2b_ref[...]


def _full(shape):
    return pl.BlockSpec(shape, lambda i: tuple(0 for _ in shape))


_roww_spec = pl.BlockSpec((BRW, 128), lambda i: (i, 0))
_sw_spec = pl.BlockSpec((NC, BRW, 128), lambda i: (0, i, 0))

_tc_head = pl.pallas_call(
    _tc_head_body,
    grid=(GRIDW,),
    in_specs=[
        _sw_spec,
        pl.BlockSpec((BRW, 16), lambda i: (i, 0)),
        _full((16, 128)),
        _full((1, 128)),
        _full((128, 128)),
    ],
    out_specs=[_roww_spec, _roww_spec],
    out_shape=[
        jax.ShapeDtypeStruct((RW, 128), jnp.float32),
        jax.ShapeDtypeStruct((RW, 128), jnp.float32),
    ],
)

_tc_mid = pl.pallas_call(
    _tc_mid_body,
    grid=(GRIDW,),
    in_specs=[_sw_spec, _roww_spec, _roww_spec, _full((1, 128)), _full((128, 128))],
    out_specs=_roww_spec,
    out_shape=jax.ShapeDtypeStruct((RW, 128), jnp.float32),
)

_tc_tail = pl.pallas_call(
    _tc_tail_body,
    grid=(GRIDW,),
    in_specs=[_sw_spec, _roww_spec, _roww_spec, _full((1, 128))],
    out_specs=_roww_spec,
    out_shape=jax.ShapeDtypeStruct((RW, 128), jnp.float32),
)

_tc_fc2 = pl.pallas_call(
    _tc_fc2_body,
    grid=(),
    in_specs=[
        pl.BlockSpec((MPAD // 4, 128), lambda: (0, 0)),
        pl.BlockSpec((128, 4), lambda: (0, 0)),
        pl.BlockSpec((1, 4), lambda: (0, 0)),
    ],
    out_specs=pl.BlockSpec((MPAD // 4, 4), lambda: (0, 0)),
    out_shape=jax.ShapeDtypeStruct((MPAD // 4, 4), jnp.float32),
)


def kernel(x, edge_index, mask_index, fc1_W, fc1_b, W1, b1, W2, b2, W3, b3,
           W4, b4, W5, b5, W6, b6, fc2_W, fc2_b):
    f32 = jnp.float32
    eye4 = jnp.eye(4, dtype=f32)
    xq = jnp.zeros((NPAD, 4), f32).at[:N].set(x).reshape(NPAD // 4, 16)
    mask_pad = jnp.zeros((MPAD,), jnp.int32).at[:M].set(mask_index)
    zeros2 = jnp.zeros((NPAD, F), f32)
    ones2 = jnp.ones((C, F), f32)

    fc1blk = jnp.kron(eye4, fc1_W)
    fc1b_w = jnp.tile(fc1_b, 4).reshape(1, 128)
    wblk = [jnp.kron(eye4, Wn) for Wn in (W1, W2, W3, W4, W5, W6)]
    bw = [jnp.tile(bn, 4).reshape(1, 128) for bn in (b1, b2, b3, b4, b5, b6)]
    fc2blk = jnp.kron(eye4, fc2_W)

    src_i = edge_index[0]
    dst_i = edge_index[1]
    deg2 = _sc_degree(dst_i, ones2, zeros2)
    deg2w = deg2.reshape(NC, RW, 128)
    dinv, p = _tc_head(deg2w, xq, fc1blk, fc1b_w, wblk[0])

    for i in range(5):
        s2 = _sc_edge_scatter(p.reshape(NPAD, F), src_i, dst_i, zeros2)
        p = _tc_mid(s2.reshape(NC, RW, 128), p, dinv, bw[i], wblk[i + 1])

    s2 = _sc_edge_scatter(p.reshape(NPAD, F), src_i, dst_i, zeros2)
    h = _tc_tail(s2.reshape(NC, RW, 128), p, dinv, bw[5])

    rows = _sc_mask_gather(h.reshape(NPAD, F), mask_pad)
    y4 = _tc_fc2(rows.reshape(MPAD // 4, 128), fc2blk, fc2_b.reshape(1, 1) * jnp.ones((1, 4), f32))
    return y4.reshape(MPAD, 1)[:M]


# speedup vs baseline: 74.0825x; 1.0217x over previous
import functools

import jax
import jax.numpy as jnp
from jax import lax
from jax.experimental import pallas as pl
from jax.experimental.pallas import tpu as pltpu
from jax.experimental.pallas import tpu_sc as plsc

N = 50000
E = 1600000
M = 10000
F = 32

NC = 2
NS = 16
NW = NC * NS

NPAD = 50048
EPT = E // NW
C = 400
NCHUNK = EPT // C
CE = 200
NCHUNKE = EPT // CE
RPT = NPAD // NS
MPAD = 10240
GPT = MPAD // NW

RW = NPAD * F // 128

_mesh = plsc.VectorSubcoreMesh(core_axis_name="c", subcore_axis_name="s")
_sc_params = pltpu.CompilerParams(use_tc_tiling_on_sc=False)


def _run_pipeline(nchunk, I, A, B, WS):
    I(0, 0)
    I(1, 1)
    A(0)

    def peel(g):
        b, b1, b2 = g % 4, (g + 1) % 4, (g + 2) % 4
        if g >= 2:
            WS(b2)
        if g + 2 < nchunk:
            I(g + 2, b2)
        if g + 1 < nchunk:
            A(b1)
        B(b)

    for g in range(4):
        peel(g)
    k = (nchunk - 2) // 4

    def body(i, carry):
        for j in range(4):
            b, b1, b2 = j, (j + 1) % 4, (j + 2) % 4
            WS(b2)
            I(4 * i + j + 2, b2)
            A(b1)
            B(b)
        return carry

    if k > 1:
        lax.fori_loop(1, k, body, 0)
    for g in range(4 * k, nchunk):
        peel(g)
    WS((nchunk - 2) % 4)
    WS((nchunk - 1) % 4)


@functools.partial(
    pl.kernel,
    out_type=jax.ShapeDtypeStruct((NC, NPAD, F), jnp.float32),
    mesh=_mesh,
    compiler_params=_sc_params,
    scratch_types=[
        [pltpu.VMEM((C,), jnp.int32) for _ in range(4)],
        pltpu.VMEM((C, F), jnp.float32),
        pltpu.VMEM_SHARED((NPAD, F), jnp.float32),
        [pltpu.SemaphoreType.DMA for _ in range(4)],
        [pltpu.SemaphoreType.DMA for _ in range(4)],
    ],
)
def _sc_degree(dst_hbm, ones_hbm, zeros_hbm, out_hbm,
               dstb, onesv, acc, sidx, ss):
    c = lax.axis_index("c")
    s = lax.axis_index("s")
    w = c * NS + s
    r0 = pl.multiple_of(s * RPT, 8)
    pltpu.sync_copy(zeros_hbm.at[pl.ds(r0, RPT)], acc.at[pl.ds(r0, RPT)])
    pltpu.sync_copy(ones_hbm, onesv)
    plsc.subcore_barrier()
    base = w * EPT

    def I(g, b):
        off = pl.multiple_of(base + g * C, 8)
        pltpu.async_copy(dst_hbm.at[pl.ds(off, C)], dstb[b], sidx[b])

    def A(b):
        pass

    def B(b):
        pltpu.make_async_copy(dst_hbm.at[pl.ds(0, C)], dstb[b], sidx[b]).wait()
        pltpu.async_copy(onesv, acc.at[dstb[b]], ss[b], add=True)

    def WS(b):
        pltpu.make_async_copy(onesv, acc.at[dstb[b]], ss[b]).wait()

    _run_pipeline(NCHUNK, I, A, B, WS)
    plsc.subcore_barrier()
    pltpu.sync_copy(acc.at[pl.ds(r0, RPT)], out_hbm.at[c, pl.ds(r0, RPT)])


NB = 4


@functools.partial(
    pl.kernel,
    out_type=jax.ShapeDtypeStruct((NC, NPAD, F), jnp.float32),
    mesh=_mesh,
    compiler_params=_sc_params,
    scratch_types=[
        [pltpu.VMEM((CE,), jnp.int32) for _ in range(NB)],
        [pltpu.VMEM((CE,), jnp.int32) for _ in range(NB)],
        [pltpu.VMEM((CE, F), jnp.float32) for _ in range(NB)],
        pltpu.VMEM_SHARED((NPAD, F), jnp.float32),
        [pltpu.SemaphoreType.DMA for _ in range(NB)],
        [pltpu.SemaphoreType.DMA for _ in range(NB)],
        [pltpu.SemaphoreType.DMA for _ in range(NB)],
    ],
)
def _sc_edge_scatter(p_hbm, src_hbm, dst_hbm, zeros_hbm, out_hbm,
                     srcb, dstb, rowsb, acc, sidx, sg, ss):
    c = lax.axis_index("c")
    s = lax.axis_index("s")
    w = c * NS + s
    r0 = pl.multiple_of(s * RPT, 8)
    pltpu.sync_copy(zeros_hbm.at[pl.ds(r0, RPT)], acc.at[pl.ds(r0, RPT)])
    plsc.subcore_barrier()
    base = w * EPT

    def I(g, b):
        off = pl.multiple_of(base + g * CE, 8)
        pltpu.async_copy(src_hbm.at[pl.ds(off, CE)], srcb[b], sidx[b])
        pltpu.async_copy(dst_hbm.at[pl.ds(off, CE)], dstb[b], sidx[b])

    def A(b):
        pltpu.make_async_copy(src_hbm.at[pl.ds(0, CE)], srcb[b], sidx[b]).wait()
        pltpu.make_async_copy(dst_hbm.at[pl.ds(0, CE)], dstb[b], sidx[b]).wait()
        pltpu.async_copy(p_hbm.at[srcb[b]], rowsb[b], sg[b])

    def B(b):
        pltpu.make_async_copy(p_hbm.at[srcb[b]], rowsb[b], sg[b]).wait()
        pltpu.async_copy(rowsb[b], acc.at[dstb[b]], ss[b], add=True)

    def WS(b):
        pltpu.make_async_copy(rowsb[b], acc.at[dstb[b]], ss[b]).wait()

    _run_pipeline(NCHUNKE, I, A, B, WS)
    plsc.subcore_barrier()
    pltpu.sync_copy(acc.at[pl.ds(r0, RPT)], out_hbm.at[c, pl.ds(r0, RPT)])


@functools.partial(
    pl.kernel,
    out_type=jax.ShapeDtypeStruct((MPAD, F), jnp.float32),
    mesh=_mesh,
    compiler_params=_sc_params,
    scratch_types=[
        pltpu.VMEM((GPT,), jnp.int32),
        pltpu.VMEM((GPT, F), jnp.float32),
        pltpu.SemaphoreType.DMA,
    ],
)
def _sc_mask_gather(h_hbm, mask_hbm, out_hbm, idxv, rowsv, sem):
    c = lax.axis_index("c")
    s = lax.axis_index("s")
    w = c * NS + s
    base = pl.multiple_of(w * GPT, 8)
    pltpu.sync_copy(mask_hbm.at[pl.ds(base, GPT)], idxv)
    pltpu.async_copy(h_hbm.at[idxv], rowsv, sem).wait()
    pltpu.sync_copy(rowsv, out_hbm.at[pl.ds(base, GPT)])


BRW = 3128
GRIDW = RW // BRW


def _tc_head_body(deg_ref, xq_ref, fc1w_ref, fc1b_ref, w1_ref,
                  dinv_ref, p_ref):
    deg = deg_ref[0] + deg_ref[1] + 1.0
    dinv = lax.rsqrt(deg)
    a = jnp.maximum(jnp.dot(xq_ref[...], fc1w_ref[...],
                            preferred_element_type=jnp.float32) + fc1b_ref[...], 0.0)
    dinv_ref[...] = dinv
    p_ref[...] = dinv * jnp.dot(a, w1_ref[...],
                                preferred_element_type=jnp.float32)


def _tc_mid_body(s_ref, p_ref, dinv_ref, b_ref, wn_ref, pn_ref):
    dinv = dinv_ref[...]
    a = jnp.maximum(dinv * (s_ref[0] + s_ref[1] + p_ref[...]) + b_ref[...], 0.0)
    pn_ref[...] = dinv * jnp.dot(a, wn_ref[...],
                                 preferred_element_type=jnp.float32)


def _tc_tail_body(s_ref, p_ref, dinv_ref, b_ref, h_ref):
    h_ref[...] = dinv_ref[...] * (s_ref[0] + s_ref[1] + p_ref[...]) + b_ref[...]


def _tc_fc2_body(rows_ref, fc2w_ref, fc2b_ref, out_ref):
    a = jnp.maximum(rows_ref[...], 0.0)
    out_ref[...] = jnp.dot(a, fc2w_ref[...],
                           preferred_element_type=jnp.float32) + fc2b_ref[...]


def _full(shape):
    return pl.BlockSpec(shape, lambda i: tuple(0 for _ in shape))


_roww_spec = pl.BlockSpec((BRW, 128), lambda i: (i, 0))
_sw_spec = pl.BlockSpec((NC, BRW, 128), lambda i: (0, i, 0))

_tc_head = pl.pallas_call(
    _tc_head_body,
    grid=(GRIDW,),
    in_specs=[
        _sw_spec,
        pl.BlockSpec((BRW, 16), lambda i: (i, 0)),
        _full((16, 128)),
        _full((1, 128)),
        _full((128, 128)),
    ],
    out_specs=[_roww_spec, _roww_spec],
    out_shape=[
        jax.ShapeDtypeStruct((RW, 128), jnp.float32),
        jax.ShapeDtypeStruct((RW, 128), jnp.float32),
    ],
)

_tc_mid = pl.pallas_call(
    _tc_mid_body,
    grid=(GRIDW,),
    in_specs=[_sw_spec, _roww_spec, _roww_spec, _full((1, 128)), _full((128, 128))],
    out_specs=_roww_spec,
    out_shape=jax.ShapeDtypeStruct((RW, 128), jnp.float32),
)

_tc_tail = pl.pallas_call(
    _tc_tail_body,
    grid=(GRIDW,),
    in_specs=[_sw_spec, _roww_spec, _roww_spec, _full((1, 128))],
    out_specs=_roww_spec,
    out_shape=jax.ShapeDtypeStruct((RW, 128), jnp.float32),
)

_tc_fc2 = pl.pallas_call(
    _tc_fc2_body,
    grid=(),
    in_specs=[
        pl.BlockSpec((MPAD // 4, 128), lambda: (0, 0)),
        pl.BlockSpec((128, 4), lambda: (0, 0)),
        pl.BlockSpec((1, 4), lambda: (0, 0)),
    ],
    out_specs=pl.BlockSpec((MPAD // 4, 4), lambda: (0, 0)),
    out_shape=jax.ShapeDtypeStruct((MPAD // 4, 4), jnp.float32),
)


def kernel(x, edge_index, mask_index, fc1_W, fc1_b, W1, b1, W2, b2, W3, b3,
           W4, b4, W5, b5, W6, b6, fc2_W, fc2_b):
    f32 = jnp.float32
    eye4 = jnp.eye(4, dtype=f32)
    xq = jnp.zeros((NPAD, 4), f32).at[:N].set(x).reshape(NPAD // 4, 16)
    mask_pad = jnp.zeros((MPAD,), jnp.int32).at[:M].set(mask_index)
    zeros2 = jnp.zeros((NPAD, F), f32)
    ones2 = jnp.ones((C, F), f32)

    fc1blk = jnp.kron(eye4, fc1_W)
    fc1b_w = jnp.tile(fc1_b, 4).reshape(1, 128)
    wblk = [jnp.kron(eye4, Wn) for Wn in (W1, W2, W3, W4, W5, W6)]
    bw = [jnp.tile(bn, 4).reshape(1, 128) for bn in (b1, b2, b3, b4, b5, b6)]
    fc2blk = jnp.kron(eye4, fc2_W)

    src_i = edge_index[0]
    dst_i = edge_index[1]
    deg2 = _sc_degree(dst_i, ones2, zeros2)
    deg2w = deg2.reshape(NC, RW, 128)
    dinv, p = _tc_head(deg2w, xq, fc1blk, fc1b_w, wblk[0])

    for i in range(5):
        s2 = _sc_edge_scatter(p.reshape(NPAD, F), src_i, dst_i, zeros2)
        p = _tc_mid(s2.reshape(NC, RW, 128), p, dinv, bw[i], wblk[i + 1])

    s2 = _sc_edge_scatter(p.reshape(NPAD, F), src_i, dst_i, zeros2)
    h = _tc_tail(s2.reshape(NC, RW, 128), p, dinv, bw[5])

    rows = _sc_mask_gather(h.reshape(NPAD, F), mask_pad)
    y4 = _tc_fc2(rows.reshape(MPAD // 4, 128), fc2blk, fc2_b.reshape(1, 1) * jnp.ones((1, 4), f32))
    return y4.reshape(MPAD, 1)[:M]
